# baseline (speedup 1.0000x reference)
import jax
import jax.numpy as jnp
from jax import lax
from jax.experimental import pallas as pl
from jax.experimental.pallas import tpu as pltpu
from jax.experimental.pallas import tpu_sc as plsc

_ROWS2 = 500000
_NC, _NS = 2, 16
_NW = _NC * _NS

_G = 128
_NPIECE = 122
_CHUNK = _G * _NPIECE
_TAIL = _ROWS2 - _NW * _CHUNK

_REL_ROWS = 1000


def _sc_gather_body(ent_in, rel_in, idx_in, ent_out, rel_out,
                    idxb, idxb32, rowb, relbuf, sg, so):
    wid = lax.axis_index("s") * _NC + lax.axis_index("c")
    base = wid * _CHUNK

    @pl.loop(0, _NPIECE)
    def _main(j):
        pltpu.sync_copy(idx_in.at[pl.ds(base + j * _G, _G)], idxb)
        pltpu.async_copy(ent_in.at[idxb], rowb, sg).wait()
        pltpu.async_copy(
            rowb, ent_out.at[pl.ds(base + j * _G, _G)], so).wait()

    @pl.when(wid == _NW - 1)
    def _copy_tail():
        tbase = _NW * _CHUNK

        @pl.loop(0, 2)
        def _tail128(k):
            pltpu.sync_copy(idx_in.at[pl.ds(tbase + k * _G, _G)], idxb)
            pltpu.async_copy(ent_in.at[idxb], rowb, sg).wait()
            pltpu.async_copy(
                rowb, ent_out.at[pl.ds(tbase + k * _G, _G)], so).wait()

        pltpu.sync_copy(idx_in.at[pl.ds(tbase + 2 * _G, 32)], idxb32)
        pltpu.async_copy(ent_in.at[idxb32], rowb.at[pl.ds(0, 32)], sg).wait()
        pltpu.async_copy(
            rowb.at[pl.ds(0, 32)],
            ent_out.at[pl.ds(tbase + 2 * _G, 32)], so).wait()

    @pl.when(wid == 0)
    def _copy_rel():
        @pl.loop(0, 7)
        def _rel128(k):
            pltpu.async_copy(
                rel_in.at[pl.ds(k * 128, 128)], relbuf, sg).wait()
            pltpu.async_copy(
                relbuf, rel_out.at[pl.ds(k * 128, 128)], so).wait()

        pltpu.async_copy(
            rel_in.at[pl.ds(7 * 128, 104)], relbuf.at[pl.ds(0, 104)], sg).wait()
        pltpu.async_copy(
            relbuf.at[pl.ds(0, 104)], rel_out.at[pl.ds(7 * 128, 104)], so).wait()


def kernel(x_dict, edge_index, entity_emb, rel_emb):
    idx = jnp.arange(_ROWS2, dtype=jnp.int32)
    ent2 = entity_emb.reshape(_ROWS2, 128)
    fn = pl.kernel(
        _sc_gather_body,
        out_type=(
            jax.ShapeDtypeStruct((_ROWS2, 128), entity_emb.dtype),
            jax.ShapeDtypeStruct(rel_emb.shape, rel_emb.dtype),
        ),
        mesh=plsc.VectorSubcoreMesh(core_axis_name="c", subcore_axis_name="s"),
        scratch_types=[
            pltpu.VMEM((_G,), jnp.int32),
            pltpu.VMEM((32,), jnp.int32),
            pltpu.VMEM((_G, 128), jnp.float32),
            pltpu.VMEM((128, 64), jnp.float32),
            pltpu.SemaphoreType.DMA,
            pltpu.SemaphoreType.DMA,
        ],
    )
    ent_out, rel_out = fn(ent2, rel_emb, idx)
    return (ent_out.reshape(entity_emb.shape), rel_out)


# speedup vs baseline: 1.4664x; 1.4664x over previous
import jax
import jax.numpy as jnp
from jax import lax
from jax.experimental import pallas as pl
from jax.experimental.pallas import tpu as pltpu
from jax.experimental.pallas import tpu_sc as plsc

_NC, _NS = 2, 16
_NW = _NC * _NS

_ENT_ROWS = 1000000
_PIECE = 504
_NPIECE = 62
_CHUNK = _PIECE * _NPIECE
_TAIL = _ENT_ROWS - _NW * _CHUNK

_REL_ROWS = 1000
_REL_SPLIT = (0, 496, 1000)


def _sc_copy_body(ent_in, rel_in, ent_out, rel_out, buf0, buf1, sem_in, sem_out):
    wid = lax.axis_index("s") * _NC + lax.axis_index("c")
    base = wid * _CHUNK
    bufs = (buf0, buf1)

    def ent_slice(j, rows):
        return pl.ds(base + j * _PIECE, rows)

    in_h = [None, None]
    out_h = [None, None]
    in_h[0] = pltpu.async_copy(
        ent_in.at[ent_slice(0, _PIECE)], bufs[0].at[pl.ds(0, _PIECE)], sem_in)
    for j in range(_NPIECE):
        b = j % 2
        in_h[b].wait()
        if j >= 1:
            out_h[1 - b].wait()
        if j + 1 < _NPIECE:
            in_h[1 - b] = pltpu.async_copy(
                ent_in.at[ent_slice(j + 1, _PIECE)],
                bufs[1 - b].at[pl.ds(0, _PIECE)], sem_in)
        out_h[b] = pltpu.async_copy(
            bufs[b].at[pl.ds(0, _PIECE)],
            ent_out.at[ent_slice(j, _PIECE)], sem_out)
    out_h[(_NPIECE - 1) % 2].wait()

    @pl.when(wid == _NW - 1)
    def _copy_tail():
        sl = pl.ds(_NW * _CHUNK, _TAIL)
        pltpu.async_copy(ent_in.at[sl], buf0.at[pl.ds(0, _TAIL)], sem_in).wait()
        pltpu.async_copy(buf0.at[pl.ds(0, _TAIL)], ent_out.at[sl], sem_out).wait()

    for k in range(2):
        @pl.when(wid == k)
        def _copy_rel(k=k):
            lo, hi = _REL_SPLIT[k], _REL_SPLIT[k + 1]
            sl = pl.ds(lo, hi - lo)
            pltpu.async_copy(rel_in.at[sl], buf0.at[pl.ds(0, hi - lo)], sem_in).wait()
            pltpu.async_copy(buf0.at[pl.ds(0, hi - lo)], rel_out.at[sl], sem_out).wait()


def kernel(x_dict, edge_index, entity_emb, rel_emb):
    fn = pl.kernel(
        _sc_copy_body,
        out_type=(
            jax.ShapeDtypeStruct(entity_emb.shape, entity_emb.dtype),
            jax.ShapeDtypeStruct(rel_emb.shape, rel_emb.dtype),
        ),
        mesh=plsc.VectorSubcoreMesh(core_axis_name="c", subcore_axis_name="s"),
        scratch_types=[
            pltpu.VMEM((_PIECE, 64), jnp.float32),
            pltpu.VMEM((_PIECE, 64), jnp.float32),
            pltpu.SemaphoreType.DMA,
            pltpu.SemaphoreType.DMA,
        ],
    )
    ent_out, rel_out = fn(entity_emb, rel_emb)
    return (ent_out, rel_out)


# speedup vs baseline: 1.4784x; 1.0082x over previous
import jax
import jax.numpy as jnp
from jax import lax
from jax.experimental import pallas as pl
from jax.experimental.pallas import tpu as pltpu
from jax.experimental.pallas import tpu_sc as plsc

_NC, _NS = 2, 16
_NW = _NC * _NS

_ENT_ROWS = 1000000
_PIECE = 248
_NBUF = 4
_NPIECE = 126
_CHUNK = _PIECE * _NPIECE
_TAIL = _ENT_ROWS - _NW * _CHUNK

_REL_ROWS = 1000
_REL_PIECES = ((0, 248), (248, 248), (496, 248), (744, 248), (992, 8))


def _sc_copy_body(ent_in, rel_in, ent_out, rel_out,
                  buf0, buf1, buf2, buf3, sem_in, sem_out):
    wid = lax.axis_index("s") * _NC + lax.axis_index("c")
    base = wid * _CHUNK
    bufs = (buf0, buf1, buf2, buf3)

    def ent_slice(j):
        return pl.ds(base + j * _PIECE, _PIECE)

    in_h = [None] * _NBUF
    out_h = [None] * _NBUF
    for k in range(_NBUF - 1):
        in_h[k] = pltpu.async_copy(
            ent_in.at[ent_slice(k)], bufs[k].at[pl.ds(0, _PIECE)], sem_in)
    for j in range(_NPIECE):
        b = j % _NBUF
        in_h[b].wait()
        out_h[b] = pltpu.async_copy(
            bufs[b].at[pl.ds(0, _PIECE)], ent_out.at[ent_slice(j)], sem_out)
        if j >= 1:
            out_h[(j - 1) % _NBUF].wait()
        nxt = j + _NBUF - 1
        if nxt < _NPIECE:
            nb = nxt % _NBUF
            in_h[nb] = pltpu.async_copy(
                ent_in.at[ent_slice(nxt)], bufs[nb].at[pl.ds(0, _PIECE)], sem_in)
    out_h[(_NPIECE - 1) % _NBUF].wait()

    @pl.when(wid == _NW - 1)
    def _copy_tail():
        sl = pl.ds(_NW * _CHUNK, _TAIL)
        pltpu.async_copy(ent_in.at[sl], buf0.at[pl.ds(0, _TAIL)], sem_in).wait()
        pltpu.async_copy(buf0.at[pl.ds(0, _TAIL)], ent_out.at[sl], sem_out).wait()

    @pl.when(wid == 0)
    def _copy_rel():
        for lo, n in _REL_PIECES:
            sl = pl.ds(lo, n)
            pltpu.async_copy(rel_in.at[sl], buf0.at[pl.ds(0, n)], sem_in).wait()
            pltpu.async_copy(buf0.at[pl.ds(0, n)], rel_out.at[sl], sem_out).wait()


def kernel(x_dict, edge_index, entity_emb, rel_emb):
    fn = pl.kernel(
        _sc_copy_body,
        out_type=(
            jax.ShapeDtypeStruct(entity_emb.shape, entity_emb.dtype),
            jax.ShapeDtypeStruct(rel_emb.shape, rel_emb.dtype),
        ),
        mesh=plsc.VectorSubcoreMesh(core_axis_name="c", subcore_axis_name="s"),
        scratch_types=[
            pltpu.VMEM((_PIECE, 64), jnp.float32),
            pltpu.VMEM((_PIECE, 64), jnp.float32),
            pltpu.VMEM((_PIECE, 64), jnp.float32),
            pltpu.VMEM((_PIECE, 64), jnp.float32),
            pltpu.SemaphoreType.DMA,
            pltpu.SemaphoreType.DMA,
        ],
    )
    ent_out, rel_out = fn(entity_emb, rel_emb)
    return (ent_out, rel_out)
